# baseline (speedup 1.0000x reference)
import jax
import jax.numpy as jnp
from jax.experimental import pallas as pl
from jax.experimental.pallas import tpu as pltpu

K = 512
D = 4096
NCH = 8
CH = K // NCH


def _copy_body(x_hbm, o_hbm, sem):
    cps = []
    for i in range(NCH):
        cp = pltpu.make_async_copy(
            x_hbm.at[pl.ds(i * CH, CH)], o_hbm.at[pl.ds(i * CH, CH)], sem.at[i]
        )
        cp.start()
        cps.append(cp)
    for cp in cps:
        cp.wait()


def kernel(embedding_weight):
    return pl.pallas_call(
        _copy_body,
        in_specs=[pl.BlockSpec(memory_space=pltpu.MemorySpace.HBM)],
        out_specs=pl.BlockSpec(memory_space=pltpu.MemorySpace.HBM),
        out_shape=jax.ShapeDtypeStruct((K, D), jnp.float32),
        scratch_shapes=[
            pltpu.SemaphoreType.DMA((NCH,)),
        ],
    )(embedding_weight)


# speedup vs baseline: 42.1045x; 42.1045x over previous
import jax
import jax.numpy as jnp
from jax.experimental import pallas as pl
from jax.experimental.pallas import tpu as pltpu

K = 512
D = 4096
NCH = 4
CH = K // NCH


def _copy_body(x_hbm, o_hbm, buf, sin, sout):
    ins = []
    for i in range(NCH):
        cp = pltpu.make_async_copy(x_hbm.at[pl.ds(i * CH, CH)], buf.at[i], sin.at[i])
        cp.start()
        ins.append(cp)
    outs = []
    for i in range(NCH):
        ins[i].wait()
        cp = pltpu.make_async_copy(buf.at[i], o_hbm.at[pl.ds(i * CH, CH)], sout.at[i])
        cp.start()
        outs.append(cp)
    for cp in outs:
        cp.wait()


def kernel(embedding_weight):
    return pl.pallas_call(
        _copy_body,
        in_specs=[pl.BlockSpec(memory_space=pltpu.MemorySpace.HBM)],
        out_specs=pl.BlockSpec(memory_space=pltpu.MemorySpace.HBM),
        out_shape=jax.ShapeDtypeStruct((K, D), jnp.float32),
        scratch_shapes=[
            pltpu.VMEM((NCH, CH, D), jnp.float32),
            pltpu.SemaphoreType.DMA((NCH,)),
            pltpu.SemaphoreType.DMA((NCH,)),
        ],
    )(embedding_weight)


# speedup vs baseline: 42.6393x; 1.0127x over previous
import jax
import jax.numpy as jnp
from jax.experimental import pallas as pl

K = 512
D = 4096
BLK = 256


def _copy_body(x_ref, o_ref):
    o_ref[...] = x_ref[...]


def kernel(embedding_weight):
    return pl.pallas_call(
        _copy_body,
        grid=(K // BLK,),
        in_specs=[pl.BlockSpec((BLK, D), lambda i: (i, 0))],
        out_specs=pl.BlockSpec((BLK, D), lambda i: (i, 0)),
        out_shape=jax.ShapeDtypeStruct((K, D), jnp.float32),
    )(embedding_weight)
